# baseline (speedup 1.0000x reference)
import functools

import jax
import jax.numpy as jnp
from jax import lax
from jax.experimental import pallas as pl
from jax.experimental.pallas import tpu as pltpu
from jax.experimental.pallas import tpu_sc as plsc

N = 10000
NP = 10240
E = 320000
EP = 331776
H = 8
NSC = 2
NTILES = 16
HPC = H // NSC
EPT = EP // NTILES
KB = 1152
KD = 96
NBC = KB // KD
NSLOT = 3
RPT = NP // NTILES
CC = 64


def _leaky(x):
    return jnp.where(x > 0, x, 0.2 * x)


_SELU_ALPHA = 1.6732632423543772848170429916717
_SELU_SCALE = 1.0507009873554804934193349852946


def _selu(x):
    return _SELU_SCALE * jnp.where(x > 0, x, _SELU_ALPHA * (jnp.exp(x) - 1.0))


def _proj_kernel(*refs, apply_selu, ncc, nin):
    nx = max(nin, 1)
    x_refs = refs[:nx]
    w_ref, a_s_ref, a_n_ref, b_ref = refs[nx:nx + 4]
    hT_refs = refs[nx + 4:nx + 4 + ncc]
    als_ref, aln_ref, gmax_ref = refs[nx + 4 + ncc:]
    i = pl.program_id(1)

    if nin == 0:
        xb = x_refs[0][...]
    else:
        parts = []
        for hh in range(x_refs[0].shape[0]):
            for r in x_refs:
                parts.append(r[hh])
        xb = jnp.concatenate(parts, axis=-1)
    if apply_selu:
        xb = _selu(xb + b_ref[0][None, :])
    als = jnp.zeros((1024,), jnp.float32)
    aln = jnp.zeros((1024,), jnp.float32)
    for cc in range(ncc):
        hcc = jnp.dot(xb, w_ref[0][:, cc * CC:(cc + 1) * CC],
                      preferred_element_type=jnp.float32)
        hT_refs[cc][...] = hcc
        als = als + jnp.sum(hcc * a_s_ref[0, 0, pl.ds(cc * CC, CC)][None, :], axis=1)
        aln = aln + jnp.sum(hcc * a_n_ref[0, 0, pl.ds(cc * CC, CC)][None, :], axis=1)
    als_ref[0, 0, pl.ds(i * 1024, 1024)] = als
    aln_ref[0, 0, pl.ds(i * 1024, 1024)] = aln
    mchunk = jnp.max(aln).reshape(1, 1)

    @pl.when(i == 0)
    def _init():
        gmax_ref[0] = mchunk

    @pl.when(i > 0)
    def _acc():
        gmax_ref[0] = jnp.maximum(gmax_ref[0], mchunk)


def _project(xs, W_T, a_self, a_neigh, b, apply_selu):
    HH, F, C = W_T.shape
    ncc = C // CC
    nb = NP // 1024
    if isinstance(xs, (list, tuple)):
        nin = len(xs)
        x_specs = [pl.BlockSpec((H, 1024, CC), lambda h, i: (0, i, 0))
                   for _ in range(nin)]
        x_args = list(xs)
    else:
        nin = 0
        if xs.shape[0] != NP:
            xs = jnp.pad(xs, ((0, NP - xs.shape[0]), (0, 0)))
        x_specs = [pl.BlockSpec((1024, F), lambda h, i: (i, 0))]
        x_args = [xs]
    outs = pl.pallas_call(
        functools.partial(_proj_kernel, apply_selu=apply_selu, ncc=ncc, nin=nin),
        grid=(HH, nb),
        in_specs=x_specs + [
            pl.BlockSpec((1, F, C), lambda h, i: (h, 0, 0)),
            pl.BlockSpec((1, 1, C), lambda h, i: (h, 0, 0)),
            pl.BlockSpec((1, 1, C), lambda h, i: (h, 0, 0)),
            pl.BlockSpec((1, F), lambda h, i: (0, 0)),
        ],
        out_specs=[
            pl.BlockSpec((1024, CC), lambda h, i, _nb=nb: (h * _nb + i, 0))
            for _ in range(ncc)
        ] + [
            pl.BlockSpec((1, 1, NP), lambda h, i: (h, 0, 0)),
            pl.BlockSpec((1, 1, NP), lambda h, i: (h, 0, 0)),
            pl.BlockSpec((1, 1, 1), lambda h, i: (h, 0, 0)),
        ],
        out_shape=[
            jax.ShapeDtypeStruct((HH * NP, CC), jnp.float32) for _ in range(ncc)
        ] + [
            jax.ShapeDtypeStruct((HH, 1, NP), jnp.float32),
            jax.ShapeDtypeStruct((HH, 1, NP), jnp.float32),
            jax.ShapeDtypeStruct((HH, 1, 1), jnp.float32),
        ],
    )(*x_args, W_T, a_self.reshape(HH, 1, C), a_neigh.reshape(HH, 1, C),
      b.reshape(1, F))
    hTs = list(outs[:ncc])
    als, aln, gmax = outs[ncc:]
    return hTs, als.reshape(HH, NP), aln.reshape(HH, NP), gmax.reshape(HH)


def _sc_layer_body(ncc, *refs):
    (src_hbm, dst_hbm) = refs[:2]
    hT_hbms = refs[2:2 + ncc]
    als_hbm, aln_hbm, gmax_hbm = refs[2 + ncc:5 + ncc]
    out_hbms = refs[5 + ncc:5 + 2 * ncc]
    rest = refs[5 + 2 * ncc:]
    (als_v, aln_v, den_v, gv_v, srcb_v, dstb_v, eeb_v) = rest[:7]
    slot_refs = rest[7:7 + 6 * NSLOT]
    slots = tuple(slot_refs[6 * i:6 * i + 6] for i in range(NSLOT))
    (zb_v, tmp_v, recsl_v, parts_s, rec_s, ee_s, acc_s) = rest[7 + 6 * NSLOT:]

    cid = lax.axis_index("c")
    sid = lax.axis_index("s")
    tile_eb = sid * EPT
    ns = sid * RPT
    zero16 = jnp.zeros((16,), jnp.float32)

    pltpu.sync_copy(gmax_hbm, gv_v.at[pl.ds(0, 16)])

    def _zb(z, _):
        for c in range(CC // 16):
            zb_v[z, pl.ds(c * 16, 16)] = zero16
        return 0
    lax.fori_loop(0, zb_v.shape[0], _zb, 0)

    def per_head(hh, _carry):
        h = cid * HPC + hh
        pltpu.sync_copy(als_hbm.at[pl.ds(h * NP, NP)], als_v)
        pltpu.sync_copy(aln_hbm.at[pl.ds(h * NP, NP)], aln_v)
        gsplat = plsc.load_gather(gv_v, [jnp.full((16,), h, jnp.int32)])

        def _zden(i, _):
            den_v[pl.ds(i * 16, 16)] = zero16
            return 0
        lax.fori_loop(0, NP // 16, _zden, 0)

        def p1_chunk(cb, _):
            base = tile_eb + cb * KB
            pltpu.sync_copy(src_hbm.at[pl.ds(base, KB)], srcb_v)
            pltpu.sync_copy(dst_hbm.at[pl.ds(base, KB)], dstb_v)

            def p1_vec(j, _):
                s16 = srcb_v[pl.ds(j * 16, 16)]
                d16 = dstb_v[pl.ds(j * 16, 16)]
                a = plsc.load_gather(aln_v, [s16])
                b = plsc.load_gather(als_v, [d16])
                ee = jnp.exp(_leaky(a + b) - _leaky(b + gsplat))
                eeb_v[pl.ds(j * 16, 16)] = ee
                plsc.addupdate_scatter(den_v, [d16], ee)
                return 0
            lax.fori_loop(0, KB // 16, p1_vec, 0)
            pltpu.sync_copy(eeb_v, ee_s.at[pl.ds(base, KB)])
            return 0
        lax.fori_loop(0, EPT // KB, p1_chunk, 0)

        pltpu.sync_copy(den_v, parts_s.at[sid])
        plsc.subcore_barrier()

        def _zrec(j, _):
            recsl_v[pl.ds(j * 16, 16)] = zero16
            return 0
        lax.fori_loop(0, RPT // 16, _zrec, 0)
        for t in range(NTILES):
            pltpu.sync_copy(parts_s.at[t, pl.ds(ns, RPT)], tmp_v)

            def _radd(j, _):
                sl = pl.ds(j * 16, 16)
                recsl_v[sl] = recsl_v[sl] + tmp_v[sl]
                return 0
            lax.fori_loop(0, RPT // 16, _radd, 0)

        def _rfin(j, _):
            sl = pl.ds(j * 16, 16)
            recsl_v[sl] = 1.0 / (recsl_v[sl] + 1e-9)
            return 0
        lax.fori_loop(0, RPT // 16, _rfin, 0)
        pltpu.sync_copy(recsl_v, rec_s.at[pl.ds(ns, RPT)])
        plsc.subcore_barrier()
        pltpu.sync_copy(rec_s, den_v)

        for cc in range(ncc):
            hT_hbm = hT_hbms[cc]
            out_hbm = out_hbms[cc]
            hbase = h * NP

            def _zacc(z, _):
                pltpu.sync_copy(zb_v, acc_s.at[pl.ds(ns + z * zb_v.shape[0],
                                                     zb_v.shape[0])])
                return 0
            lax.fori_loop(0, RPT // zb_v.shape[0], _zacc, 0)
            plsc.subcore_barrier()

            def fetch(boff, slot, wait_scatter):
                db_v, ib_v, al_v, rows_v, sem, ssem = slot

                @pl.when(wait_scatter)
                def _wsc():
                    pltpu.make_async_copy(rows_v, acc_s.at[db_v], ssem).wait()

                def bld(j, _):
                    sl = pl.ds(boff + j * 16, 16)
                    dj = pl.ds(j * 16, 16)
                    s16 = srcb_v[sl]
                    d16 = dstb_v[sl]
                    ib_v[dj] = s16 + hbase
                    db_v[dj] = d16
                    al_v[dj] = eeb_v[sl] * plsc.load_gather(den_v, [d16])
                    return 0
                lax.fori_loop(0, KD // 16, bld, 0)
                pltpu.async_copy(hT_hbm.at[ib_v], rows_v, sem)

            def finish(slot):
                db_v, ib_v, al_v, rows_v, sem, ssem = slot
                pltpu.make_async_copy(hT_hbm.at[ib_v], rows_v, sem).wait()

                def scale_row(q, _):
                    for u in range(4):
                        j = q * 4 + u
                        am = plsc.load_gather(al_v, [jnp.full((16,), j, jnp.int32)])
                        for c in range(CC // 16):
                            sl = pl.ds(c * 16, 16)
                            rows_v[j, sl] = rows_v[j, sl] * am
                    return 0
                lax.fori_loop(0, KD // 4, scale_row, 0)
                pltpu.async_copy(rows_v, acc_s.at[db_v], ssem, add=True)

            def p2_chunk(cb, _):
                base = tile_eb + cb * KB
                pltpu.sync_copy(src_hbm.at[pl.ds(base, KB)], srcb_v)
                pltpu.sync_copy(dst_hbm.at[pl.ds(base, KB)], dstb_v)
                pltpu.sync_copy(ee_s.at[pl.ds(base, KB)], eeb_v)
                armed = cb > 0
                fetch(0, slots[0], armed)
                fetch(KD, slots[1], armed)
                for k in range(NBC):
                    finish(slots[k % NSLOT])
                    if k + 2 < NBC:
                        fetch((k + 2) * KD, slots[(k + 2) % NSLOT],
                              jnp.logical_or(armed, k + 2 >= NSLOT))
                return 0
            lax.fori_loop(0, EPT // KB, p2_chunk, 0)
            for s in range(NSLOT):
                pltpu.make_async_copy(slots[s][3], acc_s.at[slots[s][0]],
                                      slots[s][5]).wait()

            plsc.subcore_barrier()
            pltpu.sync_copy(acc_s.at[pl.ds(ns, RPT)],
                            out_hbm.at[h, pl.ds(ns, RPT)])
            plsc.subcore_barrier()
        return 0

    lax.fori_loop(0, HPC, per_head, 0)


def _sc_layer(src, dst, hTs, als, aln, gmax):
    ncc = len(hTs)
    mesh = plsc.VectorSubcoreMesh(core_axis_name="c", subcore_axis_name="s")
    f32 = jnp.float32
    kern = pl.kernel(
        functools.partial(_sc_layer_body, ncc),
        out_type=[jax.ShapeDtypeStruct((H, NP, CC), f32) for _ in range(ncc)],
        mesh=mesh,
        compiler_params=pltpu.CompilerParams(needs_layout_passes=False,
                                             use_tc_tiling_on_sc=False),
        scratch_types=[
            pltpu.VMEM((NP,), f32),
            pltpu.VMEM((NP,), f32),
            pltpu.VMEM((NP,), f32),
            pltpu.VMEM((128,), f32),
            pltpu.VMEM((KB,), jnp.int32),
            pltpu.VMEM((KB,), jnp.int32),
            pltpu.VMEM((KB,), f32),
        ] + [
            st for _ in range(NSLOT) for st in (
                pltpu.VMEM((KD,), jnp.int32),
                pltpu.VMEM((KD,), jnp.int32),
                pltpu.VMEM((128,), f32),
                pltpu.VMEM((KD, CC), f32),
                pltpu.SemaphoreType.DMA,
                pltpu.SemaphoreType.DMA,
            )
        ] + [
            pltpu.VMEM((16, CC), f32),
            pltpu.VMEM((RPT,), f32),
            pltpu.VMEM((RPT,), f32),
            pltpu.VMEM_SHARED((NTILES, NP), f32),
            pltpu.VMEM_SHARED((NP,), f32),
            pltpu.VMEM_SHARED((EP,), f32),
            pltpu.VMEM_SHARED((NP, CC), f32),
        ],
    )
    outs = kern(src, dst, *hTs, als.reshape(H * NP), aln.reshape(H * NP),
                jnp.pad(gmax, (0, 16 - H)))
    return list(outs) if isinstance(outs, (list, tuple)) else [outs]


def _head_kernel(o_ref, b_ref, wd1_ref, bd1_ref, wd2_ref, bd2_ref, wo_ref, bo_ref,
                 out_ref, acc_ref):
    i = pl.program_id(0)
    nb = pl.num_programs(0)
    ob = jnp.concatenate([o_ref[hh] for hh in range(H)], axis=-1)
    h2 = _selu(ob + b_ref[0][None, :])
    part = jnp.sum(h2, axis=0, keepdims=True)

    @pl.when(i == 0)
    def _init():
        acc_ref[...] = part

    @pl.when(i > 0)
    def _acc():
        acc_ref[...] = acc_ref[...] + part

    @pl.when(i == nb - 1)
    def _final():
        g = acc_ref[...]
        d1 = _selu(jnp.dot(g, wd1_ref[...],
                           preferred_element_type=jnp.float32) + bd1_ref[0][None, :])
        d2 = _selu(jnp.dot(d1, wd2_ref[...],
                           preferred_element_type=jnp.float32) + bd2_ref[0][None, :])
        out_ref[...] = jnp.dot(d2, wo_ref[...],
                               preferred_element_type=jnp.float32) + bo_ref[0][None, :]


def _pool_head(out2, b2, Wd1, bd1, Wd2, bd2, Wo, bo):
    C2 = H * CC
    blkp = 1000
    nb = N // blkp
    return pl.pallas_call(
        _head_kernel,
        grid=(nb,),
        in_specs=[
            pl.BlockSpec((H, blkp, CC), lambda i: (0, i, 0)),
            pl.BlockSpec((1, C2), lambda i: (0, 0)),
            pl.BlockSpec(Wd1.shape, lambda i: (0, 0)),
            pl.BlockSpec((1, 64), lambda i: (0, 0)),
            pl.BlockSpec(Wd2.shape, lambda i: (0, 0)),
            pl.BlockSpec((1, 32), lambda i: (0, 0)),
            pl.BlockSpec(Wo.shape, lambda i: (0, 0)),
            pl.BlockSpec((1, 1), lambda i: (0, 0)),
        ],
        out_specs=pl.BlockSpec((1, 1), lambda i: (0, 0)),
        out_shape=jax.ShapeDtypeStruct((1, 1), jnp.float32),
        scratch_shapes=[pltpu.VMEM((1, C2), jnp.float32)],
    )(out2, b2.reshape(1, C2), Wd1, bd1.reshape(1, 64), Wd2, bd2.reshape(1, 32),
      Wo, bo.reshape(1, 1))


def kernel(x, edge_index, W1, a_self1, a_neigh1, b1, W2, a_self2, a_neigh2, b2,
           Wd1, bd1, Wd2, bd2, Wo, bo):
    pad_e = jnp.full((EP - E,), NP - 1, jnp.int32)
    src = jnp.concatenate([edge_index[0], pad_e])
    dst = jnp.concatenate([edge_index[1], pad_e])

    W1_T = jnp.transpose(W1, (1, 0, 2))
    W2_T = jnp.transpose(W2, (1, 0, 2))

    zeros_b = jnp.zeros((1, x.shape[1]), jnp.float32)
    hT1, als1, aln1, gmax1 = _project(x, W1_T, a_self1, a_neigh1, zeros_b, False)
    out1 = _sc_layer(src, dst, hT1, als1, aln1, gmax1)

    hT2, als2, aln2, gmax2 = _project(out1, W2_T, a_self2, a_neigh2, b1, True)
    out2 = _sc_layer(src, dst, hT2, als2, aln2, gmax2)

    return _pool_head(out2[0], b2, Wd1, bd1, Wd2, bd2, Wo, bo)


# speedup vs baseline: 1.1385x; 1.1385x over previous
import functools

import jax
import jax.numpy as jnp
from jax import lax
from jax.experimental import pallas as pl
from jax.experimental.pallas import tpu as pltpu
from jax.experimental.pallas import tpu_sc as plsc

N = 10000
NP = 10240
E = 320000
EP = 327680
H = 8
NSC = 2
NTILES = 16
HPC = H // NSC
EPT = EP // NTILES
KB = 1280
KD = 128
NBC = KB // KD
NSLOT = 2
RPT = NP // NTILES
CC = 64


def _leaky(x):
    return jnp.where(x > 0, x, 0.2 * x)


_SELU_ALPHA = 1.6732632423543772848170429916717
_SELU_SCALE = 1.0507009873554804934193349852946


def _selu(x):
    return _SELU_SCALE * jnp.where(x > 0, x, _SELU_ALPHA * (jnp.exp(x) - 1.0))


def _proj_kernel(*refs, apply_selu, ncc, nin):
    nx = max(nin, 1)
    x_refs = refs[:nx]
    w_ref, a_s_ref, a_n_ref, b_ref = refs[nx:nx + 4]
    hT_refs = refs[nx + 4:nx + 4 + ncc]
    als_ref, aln_ref, gmax_ref = refs[nx + 4 + ncc:]
    i = pl.program_id(1)

    if nin == 0:
        xb = x_refs[0][...]
    else:
        parts = []
        for hh in range(x_refs[0].shape[0]):
            for r in x_refs:
                parts.append(r[hh])
        xb = jnp.concatenate(parts, axis=-1)
    if apply_selu:
        xb = _selu(xb + b_ref[0][None, :])
    als = jnp.zeros((1024,), jnp.float32)
    aln = jnp.zeros((1024,), jnp.float32)
    for cc in range(ncc):
        hcc = jnp.dot(xb, w_ref[0][:, cc * CC:(cc + 1) * CC],
                      preferred_element_type=jnp.float32)
        hT_refs[cc][...] = hcc
        als = als + jnp.sum(hcc * a_s_ref[0, 0, pl.ds(cc * CC, CC)][None, :], axis=1)
        aln = aln + jnp.sum(hcc * a_n_ref[0, 0, pl.ds(cc * CC, CC)][None, :], axis=1)
    als_ref[0, 0, pl.ds(i * 1024, 1024)] = als
    aln_ref[0, 0, pl.ds(i * 1024, 1024)] = aln
    mchunk = jnp.max(aln).reshape(1, 1)

    @pl.when(i == 0)
    def _init():
        gmax_ref[0] = mchunk

    @pl.when(i > 0)
    def _acc():
        gmax_ref[0] = jnp.maximum(gmax_ref[0], mchunk)


def _project(xs, W_T, a_self, a_neigh, b, apply_selu):
    HH, F, C = W_T.shape
    ncc = C // CC
    nb = NP // 1024
    if isinstance(xs, (list, tuple)):
        nin = len(xs)
        x_specs = [pl.BlockSpec((H, 1024, CC), lambda h, i: (0, i, 0))
                   for _ in range(nin)]
        x_args = list(xs)
    else:
        nin = 0
        if xs.shape[0] != NP:
            xs = jnp.pad(xs, ((0, NP - xs.shape[0]), (0, 0)))
        x_specs = [pl.BlockSpec((1024, F), lambda h, i: (i, 0))]
        x_args = [xs]
    outs = pl.pallas_call(
        functools.partial(_proj_kernel, apply_selu=apply_selu, ncc=ncc, nin=nin),
        grid=(HH, nb),
        in_specs=x_specs + [
            pl.BlockSpec((1, F, C), lambda h, i: (h, 0, 0)),
            pl.BlockSpec((1, 1, C), lambda h, i: (h, 0, 0)),
            pl.BlockSpec((1, 1, C), lambda h, i: (h, 0, 0)),
            pl.BlockSpec((1, F), lambda h, i: (0, 0)),
        ],
        out_specs=[
            pl.BlockSpec((1024, CC), lambda h, i, _nb=nb: (h * _nb + i, 0))
            for _ in range(ncc)
        ] + [
            pl.BlockSpec((1, 1, NP), lambda h, i: (h, 0, 0)),
            pl.BlockSpec((1, 1, NP), lambda h, i: (h, 0, 0)),
            pl.BlockSpec((1, 1, 1), lambda h, i: (h, 0, 0)),
        ],
        out_shape=[
            jax.ShapeDtypeStruct((HH * NP, CC), jnp.float32) for _ in range(ncc)
        ] + [
            jax.ShapeDtypeStruct((HH, 1, NP), jnp.float32),
            jax.ShapeDtypeStruct((HH, 1, NP), jnp.float32),
            jax.ShapeDtypeStruct((HH, 1, 1), jnp.float32),
        ],
    )(*x_args, W_T, a_self.reshape(HH, 1, C), a_neigh.reshape(HH, 1, C),
      b.reshape(1, F))
    hTs = list(outs[:ncc])
    als, aln, gmax = outs[ncc:]
    return hTs, als.reshape(HH, NP), aln.reshape(HH, NP), gmax.reshape(HH)


def _sc_layer_body(ncc, *refs):
    (src_hbm, dst_hbm) = refs[:2]
    hT_hbms = refs[2:2 + ncc]
    als_hbm, aln_hbm, gmax_hbm = refs[2 + ncc:5 + ncc]
    out_hbms = refs[5 + ncc:5 + 2 * ncc]
    rest = refs[5 + 2 * ncc:]
    (als_v, aln_v, den_v, gv_v, srcb_v, dstb_v, eeb_v) = rest[:7]
    slot_refs = rest[7:7 + 6 * NSLOT]
    slots = tuple(slot_refs[6 * i:6 * i + 6] for i in range(NSLOT))
    (zb_v, tmp_v, recsl_v, parts_s, rec_s, ee_s, acc_s) = rest[7 + 6 * NSLOT:]

    cid = lax.axis_index("c")
    sid = lax.axis_index("s")
    tile_eb = sid * EPT
    ns = sid * RPT
    zero16 = jnp.zeros((16,), jnp.float32)

    pltpu.sync_copy(gmax_hbm, gv_v.at[pl.ds(0, 16)])

    def _zb(z, _):
        for c in range(CC // 16):
            zb_v[z, pl.ds(c * 16, 16)] = zero16
        return 0
    lax.fori_loop(0, zb_v.shape[0], _zb, 0)

    def per_head(hh, _carry):
        h = cid * HPC + hh
        pltpu.sync_copy(als_hbm.at[pl.ds(h * NP, NP)], als_v)
        pltpu.sync_copy(aln_hbm.at[pl.ds(h * NP, NP)], aln_v)
        gsplat = plsc.load_gather(gv_v, [jnp.full((16,), h, jnp.int32)])

        def _zden(i, _):
            den_v[pl.ds(i * 16, 16)] = zero16
            return 0
        lax.fori_loop(0, NP // 16, _zden, 0)

        def p1_chunk(cb, _):
            base = tile_eb + cb * KB
            pltpu.sync_copy(src_hbm.at[pl.ds(base, KB)], srcb_v)
            pltpu.sync_copy(dst_hbm.at[pl.ds(base, KB)], dstb_v)

            def p1_vec(j, _):
                s16 = srcb_v[pl.ds(j * 16, 16)]
                d16 = dstb_v[pl.ds(j * 16, 16)]
                a = plsc.load_gather(aln_v, [s16])
                b = plsc.load_gather(als_v, [d16])
                ee = jnp.exp(_leaky(a + b) - _leaky(b + gsplat))
                eeb_v[pl.ds(j * 16, 16)] = ee
                plsc.addupdate_scatter(den_v, [d16], ee)
                return 0
            lax.fori_loop(0, KB // 16, p1_vec, 0)
            pltpu.sync_copy(eeb_v, ee_s.at[pl.ds(base, KB)])
            return 0
        lax.fori_loop(0, EPT // KB, p1_chunk, 0)

        pltpu.sync_copy(den_v, parts_s.at[sid])
        plsc.subcore_barrier()

        def _zrec(j, _):
            recsl_v[pl.ds(j * 16, 16)] = zero16
            return 0
        lax.fori_loop(0, RPT // 16, _zrec, 0)
        for t in range(NTILES):
            pltpu.sync_copy(parts_s.at[t, pl.ds(ns, RPT)], tmp_v)

            def _radd(j, _):
                sl = pl.ds(j * 16, 16)
                recsl_v[sl] = recsl_v[sl] + tmp_v[sl]
                return 0
            lax.fori_loop(0, RPT // 16, _radd, 0)

        def _rfin(j, _):
            sl = pl.ds(j * 16, 16)
            recsl_v[sl] = 1.0 / jnp.maximum(recsl_v[sl], 1e-30)
            return 0
        lax.fori_loop(0, RPT // 16, _rfin, 0)
        pltpu.sync_copy(recsl_v, rec_s.at[pl.ds(ns, RPT)])
        plsc.subcore_barrier()
        pltpu.sync_copy(rec_s, den_v)

        for cc in range(ncc):
            hT_hbm = hT_hbms[cc]
            out_hbm = out_hbms[cc]
            hbase = h * NP

            def _zacc(z, _):
                pltpu.sync_copy(zb_v, acc_s.at[pl.ds(ns + z * zb_v.shape[0],
                                                     zb_v.shape[0])])
                return 0
            lax.fori_loop(0, RPT // zb_v.shape[0], _zacc, 0)
            plsc.subcore_barrier()

            def fetch(boff, slot):
                db_v, ib_v, al_v, rows_v, sem, _ssem = slot

                def bld(j, _):
                    sl = pl.ds(boff + j * 16, 16)
                    dj = pl.ds(j * 16, 16)
                    s16 = srcb_v[sl]
                    d16 = dstb_v[sl]
                    ib_v[dj] = s16 + hbase
                    db_v[dj] = d16
                    al_v[dj] = eeb_v[sl] * plsc.load_gather(den_v, [d16])
                    return 0
                lax.fori_loop(0, KD // 16, bld, 0)
                pltpu.async_copy(hT_hbm.at[ib_v], rows_v, sem)

            def finish(slot):
                db_v, ib_v, al_v, rows_v, sem, _ssem = slot
                pltpu.make_async_copy(hT_hbm.at[ib_v], rows_v, sem).wait()

                def scale_row(q, _):
                    for u in range(4):
                        j = q * 4 + u
                        am = plsc.load_gather(al_v, [jnp.full((16,), j, jnp.int32)])
                        for c in range(CC // 16):
                            sl = pl.ds(c * 16, 16)
                            rows_v[j, sl] = rows_v[j, sl] * am
                    return 0
                lax.fori_loop(0, KD // 4, scale_row, 0)
                pltpu.sync_copy(rows_v, acc_s.at[db_v], add=True)

            def p2_chunk(cb, _):
                base = tile_eb + cb * KB
                pltpu.sync_copy(src_hbm.at[pl.ds(base, KB)], srcb_v)
                pltpu.sync_copy(dst_hbm.at[pl.ds(base, KB)], dstb_v)
                pltpu.sync_copy(ee_s.at[pl.ds(base, KB)], eeb_v)
                fetch(0, slots[0])
                fetch(KD, slots[1])

                def pair(p, _):
                    finish(slots[0])
                    fetch((2 * p + 2) * KD, slots[0])
                    finish(slots[1])
                    fetch((2 * p + 3) * KD, slots[1])
                    return 0
                lax.fori_loop(0, NBC // 2 - 1, pair, 0)
                finish(slots[0])
                finish(slots[1])
                return 0
            lax.fori_loop(0, EPT // KB, p2_chunk, 0)

            plsc.subcore_barrier()
            pltpu.sync_copy(acc_s.at[pl.ds(ns, RPT)],
                            out_hbm.at[h, pl.ds(ns, RPT)])
            plsc.subcore_barrier()
        return 0

    lax.fori_loop(0, HPC, per_head, 0)


def _sc_layer(src, dst, hTs, als, aln, gmax):
    ncc = len(hTs)
    mesh = plsc.VectorSubcoreMesh(core_axis_name="c", subcore_axis_name="s")
    f32 = jnp.float32
    kern = pl.kernel(
        functools.partial(_sc_layer_body, ncc),
        out_type=[jax.ShapeDtypeStruct((H, NP, CC), f32) for _ in range(ncc)],
        mesh=mesh,
        compiler_params=pltpu.CompilerParams(needs_layout_passes=False,
                                             use_tc_tiling_on_sc=False),
        scratch_types=[
            pltpu.VMEM((NP,), f32),
            pltpu.VMEM((NP,), f32),
            pltpu.VMEM((NP,), f32),
            pltpu.VMEM((128,), f32),
            pltpu.VMEM((KB,), jnp.int32),
            pltpu.VMEM((KB,), jnp.int32),
            pltpu.VMEM((KB,), f32),
        ] + [
            st for _ in range(NSLOT) for st in (
                pltpu.VMEM((KD,), jnp.int32),
                pltpu.VMEM((KD,), jnp.int32),
                pltpu.VMEM((128,), f32),
                pltpu.VMEM((KD, CC), f32),
                pltpu.SemaphoreType.DMA,
                pltpu.SemaphoreType.DMA,
            )
        ] + [
            pltpu.VMEM((16, CC), f32),
            pltpu.VMEM((RPT,), f32),
            pltpu.VMEM((RPT,), f32),
            pltpu.VMEM_SHARED((NTILES, NP), f32),
            pltpu.VMEM_SHARED((NP,), f32),
            pltpu.VMEM_SHARED((EP,), f32),
            pltpu.VMEM_SHARED((NP, CC), f32),
        ],
    )
    outs = kern(src, dst, *hTs, als.reshape(H * NP), aln.reshape(H * NP),
                jnp.pad(gmax, (0, 16 - H)))
    return list(outs) if isinstance(outs, (list, tuple)) else [outs]


def _head_kernel(o_ref, b_ref, wd1_ref, bd1_ref, wd2_ref, bd2_ref, wo_ref, bo_ref,
                 out_ref, acc_ref):
    i = pl.program_id(0)
    nb = pl.num_programs(0)
    ob = jnp.concatenate([o_ref[hh] for hh in range(H)], axis=-1)
    h2 = _selu(ob + b_ref[0][None, :])
    part = jnp.sum(h2, axis=0, keepdims=True)

    @pl.when(i == 0)
    def _init():
        acc_ref[...] = part

    @pl.when(i > 0)
    def _acc():
        acc_ref[...] = acc_ref[...] + part

    @pl.when(i == nb - 1)
    def _final():
        g = acc_ref[...]
        d1 = _selu(jnp.dot(g, wd1_ref[...],
                           preferred_element_type=jnp.float32) + bd1_ref[0][None, :])
        d2 = _selu(jnp.dot(d1, wd2_ref[...],
                           preferred_element_type=jnp.float32) + bd2_ref[0][None, :])
        out_ref[...] = jnp.dot(d2, wo_ref[...],
                               preferred_element_type=jnp.float32) + bo_ref[0][None, :]


def _pool_head(out2, b2, Wd1, bd1, Wd2, bd2, Wo, bo):
    C2 = H * CC
    blkp = 1000
    nb = N // blkp
    return pl.pallas_call(
        _head_kernel,
        grid=(nb,),
        in_specs=[
            pl.BlockSpec((H, blkp, CC), lambda i: (0, i, 0)),
            pl.BlockSpec((1, C2), lambda i: (0, 0)),
            pl.BlockSpec(Wd1.shape, lambda i: (0, 0)),
            pl.BlockSpec((1, 64), lambda i: (0, 0)),
            pl.BlockSpec(Wd2.shape, lambda i: (0, 0)),
            pl.BlockSpec((1, 32), lambda i: (0, 0)),
            pl.BlockSpec(Wo.shape, lambda i: (0, 0)),
            pl.BlockSpec((1, 1), lambda i: (0, 0)),
        ],
        out_specs=pl.BlockSpec((1, 1), lambda i: (0, 0)),
        out_shape=jax.ShapeDtypeStruct((1, 1), jnp.float32),
        scratch_shapes=[pltpu.VMEM((1, C2), jnp.float32)],
    )(out2, b2.reshape(1, C2), Wd1, bd1.reshape(1, 64), Wd2, bd2.reshape(1, 32),
      Wo, bo.reshape(1, 1))


def kernel(x, edge_index, W1, a_self1, a_neigh1, b1, W2, a_self2, a_neigh2, b2,
           Wd1, bd1, Wd2, bd2, Wo, bo):
    pad_e = jnp.full((EP - E,), NP - 1, jnp.int32)
    src = jnp.concatenate([edge_index[0], pad_e])
    dst = jnp.concatenate([edge_index[1], pad_e])

    W1_T = jnp.transpose(W1, (1, 0, 2))
    W2_T = jnp.transpose(W2, (1, 0, 2))

    zeros_b = jnp.zeros((1, x.shape[1]), jnp.float32)
    hT1, als1, aln1, gmax1 = _project(x, W1_T, a_self1, a_neigh1, zeros_b, False)
    out1 = _sc_layer(src, dst, hT1, als1, aln1, gmax1)

    hT2, als2, aln2, gmax2 = _project(out1, W2_T, a_self2, a_neigh2, b1, True)
    out2 = _sc_layer(src, dst, hT2, als2, aln2, gmax2)

    return _pool_head(out2[0], b2, Wd1, bd1, Wd2, bd2, Wo, bo)


# speedup vs baseline: 1.2624x; 1.1088x over previous
import functools

import jax
import jax.numpy as jnp
from jax import lax
from jax.experimental import pallas as pl
from jax.experimental.pallas import tpu as pltpu
from jax.experimental.pallas import tpu_sc as plsc

N = 10000
NP = 10240
E = 320000
EP = 327680
H = 8
NSC = 2
NTILES = 16
HPC = H // NSC
EPT = EP // NTILES
KB = 1280
KD = 128
NBC = KB // KD
NSLOT = 2
RPT = NP // NTILES
CC = 64


def _leaky(x):
    return jnp.where(x > 0, x, 0.2 * x)


_SELU_ALPHA = 1.6732632423543772848170429916717
_SELU_SCALE = 1.0507009873554804934193349852946


def _selu(x):
    return _SELU_SCALE * jnp.where(x > 0, x, _SELU_ALPHA * (jnp.exp(x) - 1.0))


def _proj_kernel(*refs, apply_selu, ncc, nin):
    nx = max(nin, 1)
    x_refs = refs[:nx]
    w_ref, a_s_ref, a_n_ref, b_ref = refs[nx:nx + 4]
    hT_refs = refs[nx + 4:nx + 4 + ncc]
    als_ref, aln_ref, gmax_ref = refs[nx + 4 + ncc:]
    i = pl.program_id(1)

    if nin == 0:
        xb = x_refs[0][...]
    else:
        parts = []
        for hh in range(x_refs[0].shape[0]):
            for r in x_refs:
                parts.append(r[hh])
        xb = jnp.concatenate(parts, axis=-1)
    if apply_selu:
        xb = _selu(xb + b_ref[0][None, :])
    als = jnp.zeros((1024,), jnp.float32)
    aln = jnp.zeros((1024,), jnp.float32)
    for cc in range(ncc):
        hcc = jnp.dot(xb, w_ref[0][:, cc * CC:(cc + 1) * CC],
                      preferred_element_type=jnp.float32)
        hT_refs[cc][...] = hcc
        als = als + jnp.sum(hcc * a_s_ref[0, 0, pl.ds(cc * CC, CC)][None, :], axis=1)
        aln = aln + jnp.sum(hcc * a_n_ref[0, 0, pl.ds(cc * CC, CC)][None, :], axis=1)
    als_ref[0, 0, pl.ds(i * 1024, 1024)] = als
    aln_ref[0, 0, pl.ds(i * 1024, 1024)] = aln
    mchunk = jnp.max(aln).reshape(1, 1)

    @pl.when(i == 0)
    def _init():
        gmax_ref[0] = mchunk

    @pl.when(i > 0)
    def _acc():
        gmax_ref[0] = jnp.maximum(gmax_ref[0], mchunk)


def _project(xs, W_T, a_self, a_neigh, b, apply_selu):
    HH, F, C = W_T.shape
    ncc = C // CC
    nb = NP // 1024
    if isinstance(xs, (list, tuple)):
        nin = len(xs)
        x_specs = [pl.BlockSpec((H, 1024, CC), lambda h, i: (0, i, 0))
                   for _ in range(nin)]
        x_args = list(xs)
    else:
        nin = 0
        if xs.shape[0] != NP:
            xs = jnp.pad(xs, ((0, NP - xs.shape[0]), (0, 0)))
        x_specs = [pl.BlockSpec((1024, F), lambda h, i: (i, 0))]
        x_args = [xs]
    outs = pl.pallas_call(
        functools.partial(_proj_kernel, apply_selu=apply_selu, ncc=ncc, nin=nin),
        grid=(HH, nb),
        in_specs=x_specs + [
            pl.BlockSpec((1, F, C), lambda h, i: (h, 0, 0)),
            pl.BlockSpec((1, 1, C), lambda h, i: (h, 0, 0)),
            pl.BlockSpec((1, 1, C), lambda h, i: (h, 0, 0)),
            pl.BlockSpec((1, F), lambda h, i: (0, 0)),
        ],
        out_specs=[
            pl.BlockSpec((1024, CC), lambda h, i, _nb=nb: (h * _nb + i, 0))
            for _ in range(ncc)
        ] + [
            pl.BlockSpec((1, 1, NP), lambda h, i: (h, 0, 0)),
            pl.BlockSpec((1, 1, NP), lambda h, i: (h, 0, 0)),
            pl.BlockSpec((1, 1, 1), lambda h, i: (h, 0, 0)),
        ],
        out_shape=[
            jax.ShapeDtypeStruct((HH * NP, CC), jnp.float32) for _ in range(ncc)
        ] + [
            jax.ShapeDtypeStruct((HH, 1, NP), jnp.float32),
            jax.ShapeDtypeStruct((HH, 1, NP), jnp.float32),
            jax.ShapeDtypeStruct((HH, 1, 1), jnp.float32),
        ],
    )(*x_args, W_T, a_self.reshape(HH, 1, C), a_neigh.reshape(HH, 1, C),
      b.reshape(1, F))
    hTs = list(outs[:ncc])
    als, aln, gmax = outs[ncc:]
    return hTs, als.reshape(HH, NP), aln.reshape(HH, NP), gmax.reshape(HH)


def _sc_layer_body(ncc, *refs):
    (src_hbm, dst_hbm) = refs[:2]
    hT_hbms = refs[2:2 + ncc]
    als_hbm, aln_hbm, gmax_hbm = refs[2 + ncc:5 + ncc]
    out_hbms = refs[5 + ncc:5 + 2 * ncc]
    rest = refs[5 + 2 * ncc:]
    (als_v, aln_v, den_v, gv_v, srcb_v, dstb_v, eeb_v) = rest[:7]
    slot_refs = rest[7:7 + 6 * NSLOT]
    slots = tuple(slot_refs[6 * i:6 * i + 6] for i in range(NSLOT))
    (zb_v, tmp_v, recsl_v, parts_s, rec_s, ee_s, acc_s) = rest[7 + 6 * NSLOT:]

    cid = lax.axis_index("c")
    sid = lax.axis_index("s")
    tile_eb = sid * EPT
    ns = sid * RPT
    zero16 = jnp.zeros((16,), jnp.float32)

    pltpu.sync_copy(gmax_hbm, gv_v.at[pl.ds(0, 16)])

    def _zb(z, _):
        for c in range(CC // 16):
            zb_v[z, pl.ds(c * 16, 16)] = zero16
        return 0
    lax.fori_loop(0, zb_v.shape[0], _zb, 0)

    def per_head(hh, _carry):
        h = cid * HPC + hh
        pltpu.sync_copy(als_hbm.at[pl.ds(h * NP, NP)], als_v)
        pltpu.sync_copy(aln_hbm.at[pl.ds(h * NP, NP)], aln_v)
        gsplat = plsc.load_gather(gv_v, [jnp.full((16,), h, jnp.int32)])

        def _zden(i, _):
            den_v[pl.ds(i * 16, 16)] = zero16
            return 0
        lax.fori_loop(0, NP // 16, _zden, 0)

        def p1_chunk(cb, _):
            base = tile_eb + cb * KB
            pltpu.sync_copy(src_hbm.at[pl.ds(base, KB)], srcb_v)
            pltpu.sync_copy(dst_hbm.at[pl.ds(base, KB)], dstb_v)

            def p1_vec(j, _):
                s16 = srcb_v[pl.ds(j * 16, 16)]
                d16 = dstb_v[pl.ds(j * 16, 16)]
                a = plsc.load_gather(aln_v, [s16])
                b = plsc.load_gather(als_v, [d16])
                ee = jnp.exp(_leaky(a + b) - _leaky(b + gsplat))
                eeb_v[pl.ds(j * 16, 16)] = ee
                plsc.addupdate_scatter(den_v, [d16], ee)
                return 0
            lax.fori_loop(0, KB // 16, p1_vec, 0)
            pltpu.sync_copy(eeb_v, ee_s.at[pl.ds(base, KB)])
            return 0
        lax.fori_loop(0, EPT // KB, p1_chunk, 0)

        pltpu.sync_copy(den_v, parts_s.at[sid])
        plsc.subcore_barrier()

        def _zrec(j, _):
            recsl_v[pl.ds(j * 16, 16)] = zero16
            return 0
        lax.fori_loop(0, RPT // 16, _zrec, 0)
        for t in range(NTILES):
            pltpu.sync_copy(parts_s.at[t, pl.ds(ns, RPT)], tmp_v)

            def _radd(j, _):
                sl = pl.ds(j * 16, 16)
                recsl_v[sl] = recsl_v[sl] + tmp_v[sl]
                return 0
            lax.fori_loop(0, RPT // 16, _radd, 0)

        def _rfin(j, _):
            sl = pl.ds(j * 16, 16)
            recsl_v[sl] = 1.0 / jnp.maximum(recsl_v[sl], 1e-30)
            return 0
        lax.fori_loop(0, RPT // 16, _rfin, 0)
        pltpu.sync_copy(recsl_v, rec_s.at[pl.ds(ns, RPT)])
        plsc.subcore_barrier()
        pltpu.sync_copy(rec_s, den_v)

        for cc in range(ncc):
            hT_hbm = hT_hbms[cc]
            out_hbm = out_hbms[cc]
            hbase = h * NP

            def _zacc(z, _):
                pltpu.sync_copy(zb_v, acc_s.at[pl.ds(ns + z * zb_v.shape[0],
                                                     zb_v.shape[0])])
                return 0
            lax.fori_loop(0, RPT // zb_v.shape[0], _zacc, 0)
            plsc.subcore_barrier()

            def fetch(boff, slot):
                db_v, ib_v, al_v, rows_v, sem, _ssem = slot

                def bld(j, _):
                    sl = pl.ds(boff + j * 16, 16)
                    dj = pl.ds(j * 16, 16)
                    s16 = srcb_v[sl]
                    d16 = dstb_v[sl]
                    ib_v[dj] = s16 + hbase
                    db_v[dj] = d16
                    al_v[dj] = eeb_v[sl] * plsc.load_gather(den_v, [d16])
                    return 0
                lax.fori_loop(0, KD // 16, bld, 0)
                pltpu.async_copy(hT_hbm.at[ib_v], rows_v, sem)

            def finish(slot):
                db_v, ib_v, al_v, rows_v, sem, _ssem = slot
                pltpu.make_async_copy(hT_hbm.at[ib_v], rows_v, sem).wait()

                @plsc.parallel_loop(0, KD, 1, unroll=4)
                def scale_row(j):
                    am = plsc.load_gather(al_v, [jnp.full((16,), j, jnp.int32)])
                    for c in range(CC // 16):
                        sl = pl.ds(c * 16, 16)
                        rows_v[j, sl] = rows_v[j, sl] * am
                pltpu.sync_copy(rows_v, acc_s.at[db_v], add=True)

            def p2_chunk(cb, _):
                base = tile_eb + cb * KB
                pltpu.sync_copy(src_hbm.at[pl.ds(base, KB)], srcb_v)
                pltpu.sync_copy(dst_hbm.at[pl.ds(base, KB)], dstb_v)
                pltpu.sync_copy(ee_s.at[pl.ds(base, KB)], eeb_v)
                fetch(0, slots[0])
                fetch(KD, slots[1])

                def pair(p, _):
                    finish(slots[0])
                    fetch((2 * p + 2) * KD, slots[0])
                    finish(slots[1])
                    fetch((2 * p + 3) * KD, slots[1])
                    return 0
                lax.fori_loop(0, NBC // 2 - 1, pair, 0)
                finish(slots[0])
                finish(slots[1])
                return 0
            lax.fori_loop(0, EPT // KB, p2_chunk, 0)

            plsc.subcore_barrier()
            pltpu.sync_copy(acc_s.at[pl.ds(ns, RPT)],
                            out_hbm.at[h, pl.ds(ns, RPT)])
            plsc.subcore_barrier()
        return 0

    lax.fori_loop(0, HPC, per_head, 0)


def _sc_layer(src, dst, hTs, als, aln, gmax):
    ncc = len(hTs)
    mesh = plsc.VectorSubcoreMesh(core_axis_name="c", subcore_axis_name="s")
    f32 = jnp.float32
    kern = pl.kernel(
        functools.partial(_sc_layer_body, ncc),
        out_type=[jax.ShapeDtypeStruct((H, NP, CC), f32) for _ in range(ncc)],
        mesh=mesh,
        compiler_params=pltpu.CompilerParams(needs_layout_passes=False,
                                             use_tc_tiling_on_sc=False),
        scratch_types=[
            pltpu.VMEM((NP,), f32),
            pltpu.VMEM((NP,), f32),
            pltpu.VMEM((NP,), f32),
            pltpu.VMEM((128,), f32),
            pltpu.VMEM((KB,), jnp.int32),
            pltpu.VMEM((KB,), jnp.int32),
            pltpu.VMEM((KB,), f32),
        ] + [
            st for _ in range(NSLOT) for st in (
                pltpu.VMEM((KD,), jnp.int32),
                pltpu.VMEM((KD,), jnp.int32),
                pltpu.VMEM((128,), f32),
                pltpu.VMEM((KD, CC), f32),
                pltpu.SemaphoreType.DMA,
                pltpu.SemaphoreType.DMA,
            )
        ] + [
            pltpu.VMEM((16, CC), f32),
            pltpu.VMEM((RPT,), f32),
            pltpu.VMEM((RPT,), f32),
            pltpu.VMEM_SHARED((NTILES, NP), f32),
            pltpu.VMEM_SHARED((NP,), f32),
            pltpu.VMEM_SHARED((EP,), f32),
            pltpu.VMEM_SHARED((NP, CC), f32),
        ],
    )
    outs = kern(src, dst, *hTs, als.reshape(H * NP), aln.reshape(H * NP),
                jnp.pad(gmax, (0, 16 - H)))
    return list(outs) if isinstance(outs, (list, tuple)) else [outs]


def _head_kernel(o_ref, b_ref, wd1_ref, bd1_ref, wd2_ref, bd2_ref, wo_ref, bo_ref,
                 out_ref, acc_ref):
    i = pl.program_id(0)
    nb = pl.num_programs(0)
    ob = jnp.concatenate([o_ref[hh] for hh in range(H)], axis=-1)
    h2 = _selu(ob + b_ref[0][None, :])
    part = jnp.sum(h2, axis=0, keepdims=True)

    @pl.when(i == 0)
    def _init():
        acc_ref[...] = part

    @pl.when(i > 0)
    def _acc():
        acc_ref[...] = acc_ref[...] + part

    @pl.when(i == nb - 1)
    def _final():
        g = acc_ref[...]
        d1 = _selu(jnp.dot(g, wd1_ref[...],
                           preferred_element_type=jnp.float32) + bd1_ref[0][None, :])
        d2 = _selu(jnp.dot(d1, wd2_ref[...],
                           preferred_element_type=jnp.float32) + bd2_ref[0][None, :])
        out_ref[...] = jnp.dot(d2, wo_ref[...],
                               preferred_element_type=jnp.float32) + bo_ref[0][None, :]


def _pool_head(out2, b2, Wd1, bd1, Wd2, bd2, Wo, bo):
    C2 = H * CC
    blkp = 1000
    nb = N // blkp
    return pl.pallas_call(
        _head_kernel,
        grid=(nb,),
        in_specs=[
            pl.BlockSpec((H, blkp, CC), lambda i: (0, i, 0)),
            pl.BlockSpec((1, C2), lambda i: (0, 0)),
            pl.BlockSpec(Wd1.shape, lambda i: (0, 0)),
            pl.BlockSpec((1, 64), lambda i: (0, 0)),
            pl.BlockSpec(Wd2.shape, lambda i: (0, 0)),
            pl.BlockSpec((1, 32), lambda i: (0, 0)),
            pl.BlockSpec(Wo.shape, lambda i: (0, 0)),
            pl.BlockSpec((1, 1), lambda i: (0, 0)),
        ],
        out_specs=pl.BlockSpec((1, 1), lambda i: (0, 0)),
        out_shape=jax.ShapeDtypeStruct((1, 1), jnp.float32),
        scratch_shapes=[pltpu.VMEM((1, C2), jnp.float32)],
    )(out2, b2.reshape(1, C2), Wd1, bd1.reshape(1, 64), Wd2, bd2.reshape(1, 32),
      Wo, bo.reshape(1, 1))


def kernel(x, edge_index, W1, a_self1, a_neigh1, b1, W2, a_self2, a_neigh2, b2,
           Wd1, bd1, Wd2, bd2, Wo, bo):
    pad_e = jnp.full((EP - E,), NP - 1, jnp.int32)
    src = jnp.concatenate([edge_index[0], pad_e])
    dst = jnp.concatenate([edge_index[1], pad_e])

    W1_T = jnp.transpose(W1, (1, 0, 2))
    W2_T = jnp.transpose(W2, (1, 0, 2))

    zeros_b = jnp.zeros((1, x.shape[1]), jnp.float32)
    hT1, als1, aln1, gmax1 = _project(x, W1_T, a_self1, a_neigh1, zeros_b, False)
    out1 = _sc_layer(src, dst, hT1, als1, aln1, gmax1)

    hT2, als2, aln2, gmax2 = _project(out1, W2_T, a_self2, a_neigh2, b1, True)
    out2 = _sc_layer(src, dst, hT2, als2, aln2, gmax2)

    return _pool_head(out2[0], b2, Wd1, bd1, Wd2, bd2, Wo, bo)


# speedup vs baseline: 1.3112x; 1.0387x over previous
import functools

import jax
import jax.numpy as jnp
from jax import lax
from jax.experimental import pallas as pl
from jax.experimental.pallas import tpu as pltpu
from jax.experimental.pallas import tpu_sc as plsc

N = 10000
NP = 10240
E = 320000
EP = 327680
H = 8
NSC = 2
NTILES = 16
HPC = H // NSC
EPT = EP // NTILES
KB = 1280
KD = 128
NBC = KB // KD
NSLOT = 2
RPT = NP // NTILES
CC = 64


def _leaky(x):
    return jnp.where(x > 0, x, 0.2 * x)


_SELU_ALPHA = 1.6732632423543772848170429916717
_SELU_SCALE = 1.0507009873554804934193349852946


def _selu(x):
    return _SELU_SCALE * jnp.where(x > 0, x, _SELU_ALPHA * (jnp.exp(x) - 1.0))


def _proj_kernel(*refs, apply_selu, ncc, nin):
    nx = max(nin, 1)
    x_refs = refs[:nx]
    w_ref, a_s_ref, a_n_ref, b_ref = refs[nx:nx + 4]
    hT_refs = refs[nx + 4:nx + 4 + ncc]
    als_ref, aln_ref, gmax_ref = refs[nx + 4 + ncc:]
    i = pl.program_id(1)

    if nin == 0:
        xb = x_refs[0][...]
    else:
        parts = []
        for hh in range(x_refs[0].shape[0]):
            for r in x_refs:
                parts.append(r[hh])
        xb = jnp.concatenate(parts, axis=-1)
    if apply_selu:
        xb = _selu(xb + b_ref[0][None, :])
    als = jnp.zeros((1024,), jnp.float32)
    aln = jnp.zeros((1024,), jnp.float32)
    for cc in range(ncc):
        hcc = jnp.dot(xb, w_ref[0][:, cc * CC:(cc + 1) * CC],
                      preferred_element_type=jnp.float32)
        hT_refs[cc][...] = hcc
        als = als + jnp.sum(hcc * a_s_ref[0, 0, pl.ds(cc * CC, CC)][None, :], axis=1)
        aln = aln + jnp.sum(hcc * a_n_ref[0, 0, pl.ds(cc * CC, CC)][None, :], axis=1)
    als_ref[0, 0, pl.ds(i * 1024, 1024)] = als
    aln_ref[0, 0, pl.ds(i * 1024, 1024)] = aln
    mchunk = jnp.max(aln).reshape(1, 1)

    @pl.when(i == 0)
    def _init():
        gmax_ref[0] = mchunk

    @pl.when(i > 0)
    def _acc():
        gmax_ref[0] = jnp.maximum(gmax_ref[0], mchunk)


def _project(xs, W_T, a_self, a_neigh, b, apply_selu):
    HH, F, C = W_T.shape
    ncc = C // CC
    nb = NP // 1024
    if isinstance(xs, (list, tuple)):
        nin = len(xs)
        x_specs = [pl.BlockSpec((H, 1024, CC), lambda h, i: (0, i, 0))
                   for _ in range(nin)]
        x_args = list(xs)
    else:
        nin = 0
        if xs.shape[0] != NP:
            xs = jnp.pad(xs, ((0, NP - xs.shape[0]), (0, 0)))
        x_specs = [pl.BlockSpec((1024, F), lambda h, i: (i, 0))]
        x_args = [xs]
    outs = pl.pallas_call(
        functools.partial(_proj_kernel, apply_selu=apply_selu, ncc=ncc, nin=nin),
        grid=(HH, nb),
        in_specs=x_specs + [
            pl.BlockSpec((1, F, C), lambda h, i: (h, 0, 0)),
            pl.BlockSpec((1, 1, C), lambda h, i: (h, 0, 0)),
            pl.BlockSpec((1, 1, C), lambda h, i: (h, 0, 0)),
            pl.BlockSpec((1, F), lambda h, i: (0, 0)),
        ],
        out_specs=[
            pl.BlockSpec((1024, CC), lambda h, i, _nb=nb: (h * _nb + i, 0))
            for _ in range(ncc)
        ] + [
            pl.BlockSpec((1, 1, NP), lambda h, i: (h, 0, 0)),
            pl.BlockSpec((1, 1, NP), lambda h, i: (h, 0, 0)),
            pl.BlockSpec((1, 1, 1), lambda h, i: (h, 0, 0)),
        ],
        out_shape=[
            jax.ShapeDtypeStruct((HH * NP, CC), jnp.float32) for _ in range(ncc)
        ] + [
            jax.ShapeDtypeStruct((HH, 1, NP), jnp.float32),
            jax.ShapeDtypeStruct((HH, 1, NP), jnp.float32),
            jax.ShapeDtypeStruct((HH, 1, 1), jnp.float32),
        ],
    )(*x_args, W_T, a_self.reshape(HH, 1, C), a_neigh.reshape(HH, 1, C),
      b.reshape(1, F))
    hTs = list(outs[:ncc])
    als, aln, gmax = outs[ncc:]
    return hTs, als.reshape(HH, NP), aln.reshape(HH, NP), gmax.reshape(HH)


def _sc_layer_body(ncc, *refs):
    (src_hbm, dst_hbm) = refs[:2]
    hT_hbms = refs[2:2 + ncc]
    als_hbm, aln_hbm, gmax_hbm = refs[2 + ncc:5 + ncc]
    out_hbms = refs[5 + ncc:5 + 2 * ncc]
    rest = refs[5 + 2 * ncc:]
    (als_v, aln_v, den_v, gv_v, srcb_v, dstb_v, eeb_v) = rest[:7]
    slot_refs = rest[7:7 + 6 * NSLOT]
    slots = tuple(slot_refs[6 * i:6 * i + 6] for i in range(NSLOT))
    (zb_v, tmp_v, recsl_v, parts_s, rec_s, ee_s, acc_s) = rest[7 + 6 * NSLOT:]

    cid = lax.axis_index("c")
    sid = lax.axis_index("s")
    tile_eb = sid * EPT
    ns = sid * RPT
    zero16 = jnp.zeros((16,), jnp.float32)

    pltpu.sync_copy(gmax_hbm, gv_v.at[pl.ds(0, 16)])

    def _zb(z, _):
        for c in range(CC // 16):
            zb_v[z, pl.ds(c * 16, 16)] = zero16
        return 0
    lax.fori_loop(0, zb_v.shape[0], _zb, 0)

    def per_head(hh, _carry):
        h = cid * HPC + hh
        pltpu.sync_copy(als_hbm.at[pl.ds(h * NP, NP)], als_v)
        pltpu.sync_copy(aln_hbm.at[pl.ds(h * NP, NP)], aln_v)
        gsplat = plsc.load_gather(gv_v, [jnp.full((16,), h, jnp.int32)])

        def _zden(i, _):
            den_v[pl.ds(i * 16, 16)] = zero16
            return 0
        lax.fori_loop(0, NP // 16, _zden, 0)

        def p1_chunk(cb, _):
            base = tile_eb + cb * KB
            pltpu.sync_copy(src_hbm.at[pl.ds(base, KB)], srcb_v)
            pltpu.sync_copy(dst_hbm.at[pl.ds(base, KB)], dstb_v)

            @plsc.parallel_loop(0, KB // 16, 1, unroll=2)
            def p1_vec(j):
                s16 = srcb_v[pl.ds(j * 16, 16)]
                d16 = dstb_v[pl.ds(j * 16, 16)]
                a = plsc.load_gather(aln_v, [s16])
                b = plsc.load_gather(als_v, [d16])
                ee = jnp.exp(_leaky(a + b) - _leaky(b + gsplat))
                eeb_v[pl.ds(j * 16, 16)] = ee
                plsc.addupdate_scatter(den_v, [d16], ee)
            pltpu.sync_copy(eeb_v, ee_s.at[pl.ds(base, KB)])
            return 0
        lax.fori_loop(0, EPT // KB, p1_chunk, 0)

        pltpu.sync_copy(den_v, parts_s.at[sid])
        plsc.subcore_barrier()

        def _zrec(j, _):
            recsl_v[pl.ds(j * 16, 16)] = zero16
            return 0
        lax.fori_loop(0, RPT // 16, _zrec, 0)
        for t in range(NTILES):
            pltpu.sync_copy(parts_s.at[t, pl.ds(ns, RPT)], tmp_v)

            def _radd(j, _):
                sl = pl.ds(j * 16, 16)
                recsl_v[sl] = recsl_v[sl] + tmp_v[sl]
                return 0
            lax.fori_loop(0, RPT // 16, _radd, 0)

        def _rfin(j, _):
            sl = pl.ds(j * 16, 16)
            recsl_v[sl] = 1.0 / jnp.maximum(recsl_v[sl], 1e-30)
            return 0
        lax.fori_loop(0, RPT // 16, _rfin, 0)
        pltpu.sync_copy(recsl_v, rec_s.at[pl.ds(ns, RPT)])
        plsc.subcore_barrier()
        pltpu.sync_copy(rec_s, den_v)

        for cc in range(ncc):
            hT_hbm = hT_hbms[cc]
            out_hbm = out_hbms[cc]
            hbase = h * NP

            def _zacc(z, _):
                pltpu.sync_copy(zb_v, acc_s.at[pl.ds(ns + z * zb_v.shape[0],
                                                     zb_v.shape[0])])
                return 0
            lax.fori_loop(0, RPT // zb_v.shape[0], _zacc, 0)
            plsc.subcore_barrier()

            def fetch(boff, slot):
                db_v, ib_v, al_v, rows_v, sem, _ssem = slot

                @plsc.parallel_loop(0, KD // 16, 1, unroll=2)
                def bld(j):
                    sl = pl.ds(boff + j * 16, 16)
                    dj = pl.ds(j * 16, 16)
                    s16 = srcb_v[sl]
                    d16 = dstb_v[sl]
                    ib_v[dj] = s16 + hbase
                    db_v[dj] = d16
                    al_v[dj] = eeb_v[sl] * plsc.load_gather(den_v, [d16])
                pltpu.async_copy(hT_hbm.at[ib_v], rows_v, sem)

            def finish(slot):
                db_v, ib_v, al_v, rows_v, sem, _ssem = slot
                pltpu.make_async_copy(hT_hbm.at[ib_v], rows_v, sem).wait()

                @plsc.parallel_loop(0, KD, 1, unroll=4)
                def scale_row(j):
                    am = plsc.load_gather(al_v, [jnp.full((16,), j, jnp.int32)])
                    for c in range(CC // 16):
                        sl = pl.ds(c * 16, 16)
                        rows_v[j, sl] = rows_v[j, sl] * am
                pltpu.sync_copy(rows_v, acc_s.at[db_v], add=True)

            def p2_chunk(cb, _):
                base = tile_eb + cb * KB
                pltpu.sync_copy(src_hbm.at[pl.ds(base, KB)], srcb_v)
                pltpu.sync_copy(dst_hbm.at[pl.ds(base, KB)], dstb_v)
                pltpu.sync_copy(ee_s.at[pl.ds(base, KB)], eeb_v)
                fetch(0, slots[0])
                fetch(KD, slots[1])

                def pair(p, _):
                    finish(slots[0])
                    fetch((2 * p + 2) * KD, slots[0])
                    finish(slots[1])
                    fetch((2 * p + 3) * KD, slots[1])
                    return 0
                lax.fori_loop(0, NBC // 2 - 1, pair, 0)
                finish(slots[0])
                finish(slots[1])
                return 0
            lax.fori_loop(0, EPT // KB, p2_chunk, 0)

            plsc.subcore_barrier()
            pltpu.sync_copy(acc_s.at[pl.ds(ns, RPT)],
                            out_hbm.at[h, pl.ds(ns, RPT)])
            plsc.subcore_barrier()
        return 0

    lax.fori_loop(0, HPC, per_head, 0)


def _sc_layer(src, dst, hTs, als, aln, gmax):
    ncc = len(hTs)
    mesh = plsc.VectorSubcoreMesh(core_axis_name="c", subcore_axis_name="s")
    f32 = jnp.float32
    kern = pl.kernel(
        functools.partial(_sc_layer_body, ncc),
        out_type=[jax.ShapeDtypeStruct((H, NP, CC), f32) for _ in range(ncc)],
        mesh=mesh,
        compiler_params=pltpu.CompilerParams(needs_layout_passes=False,
                                             use_tc_tiling_on_sc=False),
        scratch_types=[
            pltpu.VMEM((NP,), f32),
            pltpu.VMEM((NP,), f32),
            pltpu.VMEM((NP,), f32),
            pltpu.VMEM((128,), f32),
            pltpu.VMEM((KB,), jnp.int32),
            pltpu.VMEM((KB,), jnp.int32),
            pltpu.VMEM((KB,), f32),
        ] + [
            st for _ in range(NSLOT) for st in (
                pltpu.VMEM((KD,), jnp.int32),
                pltpu.VMEM((KD,), jnp.int32),
                pltpu.VMEM((128,), f32),
                pltpu.VMEM((KD, CC), f32),
                pltpu.SemaphoreType.DMA,
                pltpu.SemaphoreType.DMA,
            )
        ] + [
            pltpu.VMEM((16, CC), f32),
            pltpu.VMEM((RPT,), f32),
            pltpu.VMEM((RPT,), f32),
            pltpu.VMEM_SHARED((NTILES, NP), f32),
            pltpu.VMEM_SHARED((NP,), f32),
            pltpu.VMEM_SHARED((EP,), f32),
            pltpu.VMEM_SHARED((NP, CC), f32),
        ],
    )
    outs = kern(src, dst, *hTs, als.reshape(H * NP), aln.reshape(H * NP),
                jnp.pad(gmax, (0, 16 - H)))
    return list(outs) if isinstance(outs, (list, tuple)) else [outs]


def _head_kernel(o_ref, b_ref, wd1_ref, bd1_ref, wd2_ref, bd2_ref, wo_ref, bo_ref,
                 out_ref, acc_ref):
    i = pl.program_id(0)
    nb = pl.num_programs(0)
    ob = jnp.concatenate([o_ref[hh] for hh in range(H)], axis=-1)
    h2 = _selu(ob + b_ref[0][None, :])
    part = jnp.sum(h2, axis=0, keepdims=True)

    @pl.when(i == 0)
    def _init():
        acc_ref[...] = part

    @pl.when(i > 0)
    def _acc():
        acc_ref[...] = acc_ref[...] + part

    @pl.when(i == nb - 1)
    def _final():
        g = acc_ref[...]
        d1 = _selu(jnp.dot(g, wd1_ref[...],
                           preferred_element_type=jnp.float32) + bd1_ref[0][None, :])
        d2 = _selu(jnp.dot(d1, wd2_ref[...],
                           preferred_element_type=jnp.float32) + bd2_ref[0][None, :])
        out_ref[...] = jnp.dot(d2, wo_ref[...],
                               preferred_element_type=jnp.float32) + bo_ref[0][None, :]


def _pool_head(out2, b2, Wd1, bd1, Wd2, bd2, Wo, bo):
    C2 = H * CC
    blkp = 1000
    nb = N // blkp
    return pl.pallas_call(
        _head_kernel,
        grid=(nb,),
        in_specs=[
            pl.BlockSpec((H, blkp, CC), lambda i: (0, i, 0)),
            pl.BlockSpec((1, C2), lambda i: (0, 0)),
            pl.BlockSpec(Wd1.shape, lambda i: (0, 0)),
            pl.BlockSpec((1, 64), lambda i: (0, 0)),
            pl.BlockSpec(Wd2.shape, lambda i: (0, 0)),
            pl.BlockSpec((1, 32), lambda i: (0, 0)),
            pl.BlockSpec(Wo.shape, lambda i: (0, 0)),
            pl.BlockSpec((1, 1), lambda i: (0, 0)),
        ],
        out_specs=pl.BlockSpec((1, 1), lambda i: (0, 0)),
        out_shape=jax.ShapeDtypeStruct((1, 1), jnp.float32),
        scratch_shapes=[pltpu.VMEM((1, C2), jnp.float32)],
    )(out2, b2.reshape(1, C2), Wd1, bd1.reshape(1, 64), Wd2, bd2.reshape(1, 32),
      Wo, bo.reshape(1, 1))


def kernel(x, edge_index, W1, a_self1, a_neigh1, b1, W2, a_self2, a_neigh2, b2,
           Wd1, bd1, Wd2, bd2, Wo, bo):
    pad_e = jnp.full((EP - E,), NP - 1, jnp.int32)
    src = jnp.concatenate([edge_index[0], pad_e])
    dst = jnp.concatenate([edge_index[1], pad_e])

    W1_T = jnp.transpose(W1, (1, 0, 2))
    W2_T = jnp.transpose(W2, (1, 0, 2))

    zeros_b = jnp.zeros((1, x.shape[1]), jnp.float32)
    hT1, als1, aln1, gmax1 = _project(x, W1_T, a_self1, a_neigh1, zeros_b, False)
    out1 = _sc_layer(src, dst, hT1, als1, aln1, gmax1)

    hT2, als2, aln2, gmax2 = _project(out1, W2_T, a_self2, a_neigh2, b1, True)
    out2 = _sc_layer(src, dst, hT2, als2, aln2, gmax2)

    return _pool_head(out2[0], b2, Wd1, bd1, Wd2, bd2, Wo, bo)
